# baseline (device time: 11518 ns/iter reference)
import jax
import jax.numpy as jnp
from jax import lax
from jax.experimental import pallas as pl
from jax.experimental.pallas import tpu as pltpu

N_DEV = 4


def kernel(x, w_mat):
    m_glob, k_per = x.shape
    k_glob, n = w_mat.shape
    m_per = m_glob // N_DEV

    def body(x_ref, w_ref, out_ref, stage_ref, comm_ref, w_bf_ref,
             send_sems, recv_sems):
        my = lax.axis_index("i")

        barrier_sem = pltpu.get_barrier_semaphore()
        for d in range(1, N_DEV):
            peer = (my + d) % N_DEV
            pl.semaphore_signal(
                barrier_sem, inc=1,
                device_id=(peer,), device_id_type=pl.DeviceIdType.MESH,
            )
        pl.semaphore_wait(barrier_sem, N_DEV - 1)

        rdmas = []
        for d in (1, 2, 3):
            target = (my + d) % N_DEV
            stage_ref[d - 1, :, :] = x_ref[
                pl.ds(target * m_per, m_per), :
            ].astype(jnp.bfloat16)
            rdma = pltpu.make_async_remote_copy(
                src_ref=stage_ref.at[d - 1],
                dst_ref=comm_ref.at[d - 1],
                send_sem=send_sems.at[d - 1],
                recv_sem=recv_sems.at[d - 1],
                device_id=(target,),
                device_id_type=pl.DeviceIdType.MESH,
            )
            rdma.start()
            rdmas.append(rdma)
        rdma_by_d = {1: rdmas[0], 2: rdmas[1], 3: rdmas[2]}

        w_bf_ref[:, :] = w_ref[:, :].astype(jnp.bfloat16)
        x_local = x_ref[pl.ds(my * m_per, m_per), :].astype(jnp.bfloat16)
        out_ref[:, :] = jnp.dot(
            x_local, w_bf_ref[pl.ds(my * k_per, k_per), :],
            preferred_element_type=jnp.float32,
        )

        for d in (1, 3, 2):
            rdma_by_d[d].wait_recv()
            src = (my - d) % N_DEV
            out_ref[:, :] += jnp.dot(
                comm_ref[d - 1, :, :],
                w_bf_ref[pl.ds(src * k_per, k_per), :],
                preferred_element_type=jnp.float32,
            )

        out_ref[:, :] = jnp.maximum(out_ref[:, :], 0.0)

        for d in (1, 2, 3):
            rdma_by_d[d].wait_send()

    return pl.pallas_call(
        body,
        out_shape=jax.ShapeDtypeStruct((m_per, n), jnp.float32),
        in_specs=[
            pl.BlockSpec(memory_space=pltpu.VMEM),
            pl.BlockSpec(memory_space=pltpu.VMEM),
        ],
        out_specs=pl.BlockSpec(memory_space=pltpu.VMEM),
        scratch_shapes=[
            pltpu.VMEM((N_DEV - 1, m_per, k_per), jnp.bfloat16),
            pltpu.VMEM((N_DEV - 1, m_per, k_per), jnp.bfloat16),
            pltpu.VMEM((k_glob, n), jnp.bfloat16),
            pltpu.SemaphoreType.DMA((N_DEV - 1,)),
            pltpu.SemaphoreType.DMA((N_DEV - 1,)),
        ],
        compiler_params=pltpu.CompilerParams(collective_id=0),
    )(x, w_mat)
